# Initial kernel scaffold; baseline (speedup 1.0000x reference)
import jax
import jax.numpy as jnp
from jax.experimental import pallas as pl


def kernel(x, edge_index, mol_ids, W_in, W_h, W_ffn1, b_ffn1, W_out, b_out):
    raise NotImplementedError("write your pallas kernel here")


# speedup vs baseline: 4.4658x; 4.4658x over previous
import functools

import jax
import jax.numpy as jnp
from jax import lax
from jax.experimental import pallas as pl
from jax.experimental.pallas import tpu as pltpu
from jax.experimental.pallas import tpu_sc as plsc

N = 10000
E = 320000
D = 128
NMOL = 4096
DEPTH = 3

BLK = 512
N_PAD = 10240
M_PAD = 4608

NW = 32
CHUNK = 128
CPT = -(-E // (NW * CHUNK))
E_PAD = NW * CHUNK * CPT
CPT_P = -(-N // (NW * CHUNK))
EP_PAD = NW * CHUNK * CPT_P


@functools.cache
def _make_sc_agg(s_pad: int, cpt: int):
    rpt = s_pad // 16
    mesh = plsc.VectorSubcoreMesh(core_axis_name="c", subcore_axis_name="s",
                                  num_cores=2, num_subcores=16)

    @functools.partial(
        pl.kernel,
        mesh=mesh,
        out_type=jax.ShapeDtypeStruct((2 * s_pad, D), jnp.float32),
        scratch_types=[
            pltpu.VMEM((cpt, CHUNK), jnp.int32),
            pltpu.VMEM((cpt, CHUNK), jnp.int32),
            pltpu.VMEM((CHUNK, D), jnp.float32),
            pltpu.VMEM_SHARED((s_pad, D), jnp.float32),
            pltpu.SemaphoreType.DMA,
        ],
    )
    def sc_agg(g_hbm, src_hbm, dst_hbm, zeros_hbm, out_hbm,
               src_v, dst_v, buf, agg_s, sem):
        cid = lax.axis_index("c")
        sid = lax.axis_index("s")
        w = cid * 16 + sid
        pltpu.sync_copy(src_hbm.at[w], src_v)
        pltpu.sync_copy(dst_hbm.at[w], dst_v)
        pltpu.sync_copy(zeros_hbm.at[pl.ds(sid * rpt, rpt)],
                        agg_s.at[pl.ds(sid * rpt, rpt)])
        plsc.subcore_barrier()

        def body(j, carry):
            pltpu.async_copy(g_hbm.at[src_v.at[j]], buf, sem).wait()
            pltpu.sync_copy(buf, agg_s.at[dst_v.at[j]], add=True)
            return carry

        lax.fori_loop(0, cpt, body, 0)
        plsc.subcore_barrier()
        pltpu.sync_copy(agg_s.at[pl.ds(sid * rpt, rpt)],
                        out_hbm.at[pl.ds(cid * s_pad + sid * rpt, rpt)])

    return sc_agg


def _tc_h0g(x_p, W_in, W_h):
    def body(x_ref, wi_ref, wh_ref, h0_ref, g_ref):
        h0 = jnp.maximum(
            jnp.dot(x_ref[...], wi_ref[...], preferred_element_type=jnp.float32),
            0.0)
        h0_ref[...] = h0
        g_ref[...] = jnp.dot(h0, wh_ref[...], preferred_element_type=jnp.float32)

    return pl.pallas_call(
        body,
        grid=(N_PAD // BLK,),
        in_specs=[
            pl.BlockSpec((BLK, D), lambda i: (i, 0)),
            pl.BlockSpec((D, D), lambda i: (0, 0)),
            pl.BlockSpec((D, D), lambda i: (0, 0)),
        ],
        out_specs=[pl.BlockSpec((BLK, D), lambda i: (i, 0))] * 2,
        out_shape=[jax.ShapeDtypeStruct((N_PAD, D), jnp.float32)] * 2,
    )(x_p, W_in, W_h)


def _tc_round(h0, q, W_h):
    nb = N_PAD // BLK

    def body(h0_ref, q0_ref, q1_ref, wh_ref, h_ref, g_ref):
        h = jnp.maximum(h0_ref[...] + q0_ref[...] + q1_ref[...], 0.0)
        h_ref[...] = h
        g_ref[...] = jnp.dot(h, wh_ref[...], preferred_element_type=jnp.float32)

    return pl.pallas_call(
        body,
        grid=(nb,),
        in_specs=[
            pl.BlockSpec((BLK, D), lambda i: (i, 0)),
            pl.BlockSpec((BLK, D), lambda i: (i, 0)),
            pl.BlockSpec((BLK, D), lambda i: (i + nb, 0)),
            pl.BlockSpec((D, D), lambda i: (0, 0)),
        ],
        out_specs=[pl.BlockSpec((BLK, D), lambda i: (i, 0))] * 2,
        out_shape=[jax.ShapeDtypeStruct((N_PAD, D), jnp.float32)] * 2,
    )(h0, q, q, W_h)


def _tc_final(m, W_ffn1, b_ffn1, W_out_p, b_out_p):
    nb = NMOL // BLK
    off = M_PAD // BLK

    def body(m0_ref, m1_ref, w1_ref, b1_ref, wo_ref, bo_ref, out_ref):
        mv = m0_ref[...] + m1_ref[...]
        z = jnp.maximum(
            jnp.dot(mv, w1_ref[...], preferred_element_type=jnp.float32)
            + b1_ref[...], 0.0)
        out_ref[...] = (
            jnp.dot(z, wo_ref[...], preferred_element_type=jnp.float32)
            + bo_ref[...])

    return pl.pallas_call(
        body,
        grid=(nb,),
        in_specs=[
            pl.BlockSpec((BLK, D), lambda i: (i, 0)),
            pl.BlockSpec((BLK, D), lambda i: (i + off, 0)),
            pl.BlockSpec((D, D), lambda i: (0, 0)),
            pl.BlockSpec((1, D), lambda i: (0, 0)),
            pl.BlockSpec((D, D), lambda i: (0, 0)),
            pl.BlockSpec((1, D), lambda i: (0, 0)),
        ],
        out_specs=pl.BlockSpec((BLK, D), lambda i: (i, 0)),
        out_shape=jax.ShapeDtypeStruct((NMOL, D), jnp.float32),
    )(m, m, W_ffn1, b_ffn1, W_out_p, b_out_p)


def kernel(x, edge_index, mol_ids, W_in, W_h, W_ffn1, b_ffn1, W_out, b_out):
    src = edge_index[0].astype(jnp.int32)
    dst = edge_index[1].astype(jnp.int32)

    src_p = jnp.concatenate(
        [src, jnp.zeros((E_PAD - E,), jnp.int32)]).reshape(NW, CPT, CHUNK)
    dst_p = jnp.concatenate(
        [dst, jnp.full((E_PAD - E,), N, jnp.int32)]).reshape(NW, CPT, CHUNK)
    iota_p = jnp.concatenate(
        [jnp.arange(N, dtype=jnp.int32),
         jnp.zeros((EP_PAD - N,), jnp.int32)]).reshape(NW, CPT_P, CHUNK)
    mol_p = jnp.concatenate(
        [mol_ids.astype(jnp.int32),
         jnp.full((EP_PAD - N,), NMOL, jnp.int32)]).reshape(NW, CPT_P, CHUNK)
    zeros = jnp.zeros((N_PAD, D), jnp.float32)
    x_p = jnp.pad(x, ((0, N_PAD - N), (0, 0)))

    h0, g = _tc_h0g(x_p, W_in, W_h)
    sc_round = _make_sc_agg(N_PAD, CPT)
    h = h0
    for _ in range(DEPTH):
        q = sc_round(g, src_p, dst_p, zeros)
        h, g = _tc_round(h0, q, W_h)

    sc_pool = _make_sc_agg(M_PAD, CPT_P)
    m = sc_pool(h, iota_p, mol_p, zeros)

    W_out_p = jnp.pad(W_out, ((0, 0), (0, D - W_out.shape[1])))
    b_out_p = jnp.pad(b_out, (0, D - b_out.shape[0])).reshape(1, D)
    out_full = _tc_final(m, W_ffn1, b_ffn1.reshape(1, D), W_out_p, b_out_p)
    return out_full[:, :W_out.shape[1]]
